# baseline (speedup 1.0000x reference)
import jax
import jax.numpy as jnp
from jax.experimental import pallas as pl

_IN = 4096
_OUT = 32768
_K = 1638
_TILE = 512
_GRID = _OUT // _TILE

def _key_to_f32(k):
    sign = jnp.uint32(0x80000000)
    bits = jnp.where(k >= sign, k ^ sign, ~k)
    return jax.lax.bitcast_convert_type(bits, jnp.float32)


def _fused_kernel(x_ref, w1_ref, w2_ref, b_ref, o_ref):
    j = pl.program_id(0)
    dn = (((1,), (1,)), ((), ()))
    y1 = jax.lax.dot_general(x_ref[...], w1_ref[...], dimension_numbers=dn,
                             preferred_element_type=jnp.float32)
    y2 = jax.lax.dot_general(x_ref[...], w2_ref[...], dimension_numbers=dn,
                             preferred_element_type=jnp.float32)
    o_ref[:, pl.ds(j * _TILE, _TILE)] = (
        jnp.concatenate([y1, y2], axis=1) + b_ref[...])

    @pl.when(j == _GRID - 1)
    def _select():
        yf = o_ref[...]
        t = jnp.zeros((yf.shape[0], 1), jnp.uint32)
        for bit in range(31, -1, -1):
            cand = t | jnp.uint32(1 << bit)
            cnt = jnp.sum(jnp.where(yf >= _key_to_f32(cand), 1.0, 0.0),
                          axis=1, keepdims=True)
            t = jnp.where(cnt >= _K, cand, t)
        o_ref[...] = jnp.where(yf >= _key_to_f32(t), yf, 0.0)


def kernel(x, W, b):
    b2 = b.reshape(1, _OUT)
    return pl.pallas_call(
        _fused_kernel,
        grid=(_GRID,),
        in_specs=[
            pl.BlockSpec((x.shape[0], _IN), lambda j: (0, 0)),
            pl.BlockSpec((_TILE // 2, _IN), lambda j: (2 * j, 0)),
            pl.BlockSpec((_TILE // 2, _IN), lambda j: (2 * j + 1, 0)),
            pl.BlockSpec((1, _TILE), lambda j: (0, j)),
        ],
        out_specs=pl.BlockSpec((x.shape[0], _OUT), lambda j: (0, 0)),
        out_shape=jax.ShapeDtypeStruct((x.shape[0], _OUT), jnp.float32),
    )(x, W, W, b2)


# speedup vs baseline: 1.0234x; 1.0234x over previous
import jax
import jax.numpy as jnp
from jax.experimental import pallas as pl

_IN = 4096
_OUT = 32768
_K = 1638
_TILE = 512
_GRID = _OUT // _TILE

def _key_to_f32(k):
    sign = jnp.uint32(0x80000000)
    bits = jnp.where(k >= sign, k ^ sign, ~k)
    return jax.lax.bitcast_convert_type(bits, jnp.float32)


def _fused_kernel(x_ref, w1_ref, w2_ref, b_ref, o_ref):
    j = pl.program_id(0)
    dn = (((1,), (1,)), ((), ()))
    y1 = jax.lax.dot_general(x_ref[...], w1_ref[...], dimension_numbers=dn,
                             preferred_element_type=jnp.float32)
    y2 = jax.lax.dot_general(x_ref[...], w2_ref[...], dimension_numbers=dn,
                             preferred_element_type=jnp.float32)
    o_ref[:, pl.ds(j * _TILE, _TILE)] = (
        jnp.concatenate([y1, y2], axis=1) + b_ref[...])

    @pl.when(j == _GRID - 1)
    def _select():
        yf = o_ref[...]
        t = jnp.zeros((yf.shape[0], 1), jnp.uint32)
        for bit in range(31, -1, -1):
            cand = t | jnp.uint32(1 << bit)
            cf = _key_to_f32(cand)
            parts = [jnp.where(o_ref[:, i * 4096:(i + 1) * 4096] >= cf,
                               1.0, 0.0) for i in range(8)]
            while len(parts) > 1:
                parts = [parts[i] + parts[i + 1]
                         for i in range(0, len(parts), 2)]
            m = parts[0]
            while m.shape[1] > 128:
                h = m.shape[1] // 2
                m = m[:, :h] + m[:, h:]
            cnt = jnp.sum(m, axis=1, keepdims=True)
            t = jnp.where(cnt >= _K, cand, t)
        o_ref[...] = jnp.where(yf >= _key_to_f32(t), yf, 0.0)


def kernel(x, W, b):
    b2 = b.reshape(1, _OUT)
    return pl.pallas_call(
        _fused_kernel,
        grid=(_GRID,),
        in_specs=[
            pl.BlockSpec((x.shape[0], _IN), lambda j: (0, 0)),
            pl.BlockSpec((_TILE // 2, _IN), lambda j: (2 * j, 0)),
            pl.BlockSpec((_TILE // 2, _IN), lambda j: (2 * j + 1, 0)),
            pl.BlockSpec((1, _TILE), lambda j: (0, j)),
        ],
        out_specs=pl.BlockSpec((x.shape[0], _OUT), lambda j: (0, 0)),
        out_shape=jax.ShapeDtypeStruct((x.shape[0], _OUT), jnp.float32),
    )(x, W, W, b2)


# speedup vs baseline: 1.0316x; 1.0080x over previous
import jax
import jax.numpy as jnp
from jax.experimental import pallas as pl

_IN = 4096
_OUT = 32768
_K = 1638
_TILE = 512
_GRID = _OUT // _TILE

def _key_to_f32(k):
    sign = jnp.uint32(0x80000000)
    bits = jnp.where(k >= sign, k ^ sign, ~k)
    return jax.lax.bitcast_convert_type(bits, jnp.float32)


def _fused_kernel(x_ref, w1_ref, w2_ref, b_ref, o_ref):
    j = pl.program_id(0)
    dn = (((1,), (1,)), ((), ()))
    y1 = jax.lax.dot_general(x_ref[...], w1_ref[...], dimension_numbers=dn,
                             preferred_element_type=jnp.float32)
    y2 = jax.lax.dot_general(x_ref[...], w2_ref[...], dimension_numbers=dn,
                             preferred_element_type=jnp.float32)
    o_ref[:, pl.ds(j * _TILE, _TILE)] = (
        jnp.concatenate([y1, y2], axis=1) + b_ref[...])

    @pl.when(j == _GRID - 1)
    def _select():
        yf = o_ref[...]
        t = jnp.zeros((yf.shape[0], 1), jnp.uint32)
        for bit in range(31, -1, -1):
            cand = t | jnp.uint32(1 << bit)
            cf = _key_to_f32(cand)
            acc = None
            for i in range(8):
                p = jnp.where(o_ref[:, i * 4096:(i + 1) * 4096] >= cf,
                              1.0, 0.0)
                while p.shape[1] > 128:
                    h = p.shape[1] // 2
                    p = p[:, :h] + p[:, h:]
                acc = p if acc is None else acc + p
            cnt = jnp.sum(acc, axis=1, keepdims=True)
            t = jnp.where(cnt >= _K, cand, t)
        o_ref[...] = jnp.where(yf >= _key_to_f32(t), yf, 0.0)


def kernel(x, W, b):
    b2 = b.reshape(1, _OUT)
    return pl.pallas_call(
        _fused_kernel,
        grid=(_GRID,),
        in_specs=[
            pl.BlockSpec((x.shape[0], _IN), lambda j: (0, 0)),
            pl.BlockSpec((_TILE // 2, _IN), lambda j: (2 * j, 0)),
            pl.BlockSpec((_TILE // 2, _IN), lambda j: (2 * j + 1, 0)),
            pl.BlockSpec((1, _TILE), lambda j: (0, j)),
        ],
        out_specs=pl.BlockSpec((x.shape[0], _OUT), lambda j: (0, 0)),
        out_shape=jax.ShapeDtypeStruct((x.shape[0], _OUT), jnp.float32),
    )(x, W, W, b2)


# speedup vs baseline: 1.0319x; 1.0002x over previous
import jax
import jax.numpy as jnp
from jax.experimental import pallas as pl
from jax.experimental.pallas import tpu as pltpu

_IN = 4096
_OUT = 32768
_K = 1638
_TILE = 512
_GRID = _OUT // _TILE


def _f32_to_key(i):
    return i ^ ((i >> 31) & jnp.int32(0x7FFFFFFF))


def _key_to_f32(k):
    bits = k ^ ((k >> 31) & jnp.int32(0x7FFFFFFF))
    return jax.lax.bitcast_convert_type(bits, jnp.float32)


def _tree_count(parts):
    acc = None
    for p in parts:
        while p.shape[1] > 128:
            h = p.shape[1] // 2
            p = p[:, :h] + p[:, h:]
        acc = p if acc is None else acc + p
    return jnp.sum(acc.astype(jnp.int32), axis=1, keepdims=True)


def _fused_kernel(x_ref, w1_ref, w2_ref, b_ref, o_ref, hi_ref):
    j = pl.program_id(0)
    dn = (((1,), (1,)), ((), ()))
    y1 = jax.lax.dot_general(x_ref[...], w1_ref[...], dimension_numbers=dn,
                             preferred_element_type=jnp.float32)
    y2 = jax.lax.dot_general(x_ref[...], w2_ref[...], dimension_numbers=dn,
                             preferred_element_type=jnp.float32)
    y = jnp.concatenate([y1, y2], axis=1) + b_ref[...]
    o_ref[:, pl.ds(j * _TILE, _TILE)] = y
    key = _f32_to_key(jax.lax.bitcast_convert_type(y, jnp.int32))
    hi_ref[:, pl.ds(j * _TILE, _TILE)] = (key >> 16).astype(jnp.int16)

    @pl.when(j == _GRID - 1)
    def _select():
        rows = o_ref.shape[0]
        tb = jnp.zeros((rows, 1), jnp.int32)
        for bit in range(15, -1, -1):
            cand = tb | jnp.int32(1 << bit)
            c16 = (cand ^ jnp.int32(0x8000)).astype(jnp.int16)
            cnt = _tree_count(
                [jnp.where(hi_ref[:, i * 4096:(i + 1) * 4096] >= c16,
                           jnp.int16(1), jnp.int16(0)) for i in range(8)])
            tb = jnp.where(cnt >= _K, cand, tb)
        t = (tb ^ jnp.int32(0x8000)) << 16
        for bit in range(15, -1, -1):
            cand = t | jnp.int32(1 << bit)
            cf = _key_to_f32(cand)
            cnt = _tree_count(
                [jnp.where(o_ref[:, i * 4096:(i + 1) * 4096] >= cf,
                           1.0, 0.0) for i in range(8)])
            t = jnp.where(cnt >= _K, cand, t)
        yf = o_ref[...]
        o_ref[...] = jnp.where(yf >= _key_to_f32(t), yf, 0.0)


def kernel(x, W, b):
    b2 = b.reshape(1, _OUT)
    return pl.pallas_call(
        _fused_kernel,
        grid=(_GRID,),
        in_specs=[
            pl.BlockSpec((x.shape[0], _IN), lambda j: (0, 0)),
            pl.BlockSpec((_TILE // 2, _IN), lambda j: (2 * j, 0)),
            pl.BlockSpec((_TILE // 2, _IN), lambda j: (2 * j + 1, 0)),
            pl.BlockSpec((1, _TILE), lambda j: (0, j)),
        ],
        out_specs=pl.BlockSpec((x.shape[0], _OUT), lambda j: (0, 0)),
        out_shape=jax.ShapeDtypeStruct((x.shape[0], _OUT), jnp.float32),
        scratch_shapes=[pltpu.VMEM((x.shape[0], _OUT), jnp.int16)],
    )(x, W, W, b2)
